# Initial kernel scaffold; baseline (speedup 1.0000x reference)
import functools

import jax
import jax.numpy as jnp
from jax import lax
from jax.experimental import pallas as pl
from jax.experimental.pallas import tpu as pltpu
from jax.experimental.pallas import tpu_sc as plsc

N_NODES = 10000
N_HEDGES = 10000
N_EDGES = 320000
IVD = 128
VD = 128
QD = 64
ED = 128
NCLS = 40

_NC = 2
_NS = 16
_L = 16
_NW = _NC * _NS
_C = 256
_NCHUNK = N_EDGES // _C
_RPT = N_NODES // _NS
_AW = VD + 16

_mesh = plsc.VectorSubcoreMesh(
    core_axis_name="c", subcore_axis_name="s", num_cores=_NC, num_subcores=_NS
)


def _worker_id():
    return lax.axis_index("s") * _NC + lax.axis_index("c")


def _num_chunks(wid):
    return (_NCHUNK - 1 - wid) // _NW + 1


@functools.partial(
    pl.kernel,
    out_type=[
        jax.ShapeDtypeStruct((N_EDGES,), jnp.float32),
        jax.ShapeDtypeStruct((_NW, _L), jnp.float32),
    ],
    mesh=_mesh,
    scratch_types=[
        pltpu.VMEM((_C,), jnp.int32),
        pltpu.VMEM((_C,), jnp.int32),
        pltpu.VMEM((_C, QD), jnp.float32),
        pltpu.VMEM((_C, QD), jnp.float32),
        pltpu.VMEM((_C,), jnp.float32),
        pltpu.VMEM((_L,), jnp.float32),
        pltpu.SemaphoreType.DMA,
        pltpu.SemaphoreType.DMA,
    ],
)
def _edge_attn(k_hbm, q_hbm, src_hbm, dst_hbm, attn_hbm, wmax_hbm,
               srcv, dstv, kbuf, qbuf, attnv, maxv, sem0, sem1):
    wid = _worker_id()
    eidx0 = lax.iota(jnp.int32, _L)

    def chunk_body(i, macc):
        base = (wid + i * _NW) * _C
        pltpu.sync_copy(src_hbm.at[pl.ds(base, _C)], srcv)
        pltpu.sync_copy(dst_hbm.at[pl.ds(base, _C)], dstv)
        cp0 = pltpu.async_copy(k_hbm.at[srcv], kbuf, sem0)
        cp1 = pltpu.async_copy(q_hbm.at[dstv], qbuf, sem1)
        cp0.wait()
        cp1.wait()

        def grp_body(g, macc):
            e0 = g * _L
            ei = eidx0 + e0
            acc = jnp.zeros((_L,), jnp.float32)
            for d in range(QD):
                dv = jnp.full((_L,), d, jnp.int32)
                acc = acc + (plsc.load_gather(kbuf, [ei, dv]) *
                             plsc.load_gather(qbuf, [ei, dv]))
            a = jnp.where(acc >= 0.0, acc, acc * 0.01) * 0.125
            attnv[pl.ds(e0, _L)] = a
            return jnp.maximum(macc, a)

        macc = lax.fori_loop(0, _C // _L, grp_body, macc)
        pltpu.sync_copy(attnv, attn_hbm.at[pl.ds(base, _C)])
        return macc

    macc = lax.fori_loop(0, _num_chunks(wid), chunk_body,
                         jnp.full((_L,), -1e30, jnp.float32))
    maxv[...] = macc
    pltpu.sync_copy(maxv, wmax_hbm.at[wid])


@functools.partial(
    pl.kernel,
    out_type=jax.ShapeDtypeStruct((_NC, N_NODES, _AW), jnp.float32),
    mesh=_mesh,
    scratch_types=[
        pltpu.VMEM((_C,), jnp.int32),
        pltpu.VMEM((_C,), jnp.int32),
        pltpu.VMEM((_C,), jnp.float32),
        pltpu.VMEM((_C, VD), jnp.float32),
        pltpu.VMEM((_C, _AW), jnp.float32),
        pltpu.VMEM((_NW, _L), jnp.float32),
        pltpu.VMEM_SHARED((N_NODES, _AW), jnp.float32),
        pltpu.SemaphoreType.DMA,
    ],
)
def _edge_agg(v_hbm, src_hbm, dst_hbm, attn_hbm, wmax_hbm, acc_hbm,
              srcv, dstv, attnv, vbuf, msgbuf, wmaxv, accum, sem0):
    cid = lax.axis_index("c")
    sid = lax.axis_index("s")
    wid = sid * _NC + cid
    eidx0 = lax.iota(jnp.int32, _L)

    pltpu.sync_copy(wmax_hbm, wmaxv)
    m = wmaxv[0]
    for r in range(1, _NW):
        m = jnp.maximum(m, wmaxv[r])
    gmax = jnp.max(m)

    zero16 = jnp.zeros((_L,), jnp.float32)

    def zrow(e, carry):
        for j in range(_AW // _L):
            msgbuf[e, pl.ds(j * _L, _L)] = zero16
        return carry

    lax.fori_loop(0, _C, zrow, 0)
    row0 = sid * _RPT
    for off, sz in ((0, _C), (_C, _C), (2 * _C, _RPT - 2 * _C)):
        pltpu.sync_copy(msgbuf.at[pl.ds(0, sz)], accum.at[pl.ds(row0 + off, sz)])
    plsc.subcore_barrier()

    col_ex = jnp.full((_L,), VD, jnp.int32)

    def chunk_body(i, carry):
        base = (wid + i * _NW) * _C
        pltpu.sync_copy(src_hbm.at[pl.ds(base, _C)], srcv)
        pltpu.sync_copy(dst_hbm.at[pl.ds(base, _C)], dstv)
        pltpu.sync_copy(attn_hbm.at[pl.ds(base, _C)], attnv)
        cp = pltpu.async_copy(v_hbm.at[srcv], vbuf, sem0)
        cp.wait()

        def grp_body(g, carry):
            e0 = g * _L
            ei = eidx0 + e0
            ex = jnp.exp(attnv[pl.ds(e0, _L)] - gmax)
            plsc.store_scatter(msgbuf, [ei, col_ex], ex)
            for d in range(VD):
                dv = jnp.full((_L,), d, jnp.int32)
                col = plsc.load_gather(vbuf, [ei, dv]) * ex
                plsc.store_scatter(msgbuf, [ei, dv], col)
            return carry

        lax.fori_loop(0, _C // _L, grp_body, 0)
        pltpu.sync_copy(msgbuf, accum.at[dstv], add=True)
        return carry

    lax.fori_loop(0, _num_chunks(wid), chunk_body, 0)
    plsc.subcore_barrier()

    for off, sz in ((0, _C), (_C, _C), (2 * _C, _RPT - 2 * _C)):
        pltpu.sync_copy(accum.at[pl.ds(row0 + off, sz)],
                        acc_hbm.at[cid, pl.ds(row0 + off, sz)])


_ROWS = 1000


def _proj_body(vf, ef, wv1t, bv1, we1t, be1, wqvt, bqv, wket, bke, wvet, bve,
               fe_o, q_o, k_o, v_o):
    fv = jnp.dot(vf[...], wv1t[...], preferred_element_type=jnp.float32) + bv1[...]
    fe = jnp.dot(ef[...], we1t[...], preferred_element_type=jnp.float32) + be1[...]
    fe_o[...] = fe
    q_o[...] = jnp.dot(fv, wqvt[...], preferred_element_type=jnp.float32) + bqv[...]
    k_o[...] = jnp.dot(fe, wket[...], preferred_element_type=jnp.float32) + bke[...]
    v_o[...] = jnp.dot(fe, wvet[...], preferred_element_type=jnp.float32) + bve[...]


def _final_body(acc, wclst, bcls, h_o, pred_o):
    s = acc[0] + acc[1]
    denom = s[:, VD:VD + 1]
    h = s[:, :VD] / (denom + 1e-9)
    h_o[...] = h
    pred_o[...] = jnp.dot(h, wclst[...], preferred_element_type=jnp.float32) + bcls[...]


def _row_spec(cols):
    return pl.BlockSpec((_ROWS, cols), lambda i: (i, 0))


def _full_spec(shape):
    nd = len(shape)
    return pl.BlockSpec(shape, lambda i: (0,) * nd)


def kernel(vfeat, efeat, edge_index, W_v1, b_v1, W_e1, b_e1, W_qv, b_qv,
           W_ke, b_ke, W_ve, b_ve, W_cls, b_cls, first_layer, last_layer):
    src = edge_index[0]
    dst = edge_index[1]
    grid = N_NODES // _ROWS

    feat_e, q, k, v = pl.pallas_call(
        _proj_body,
        grid=(grid,),
        in_specs=[
            _row_spec(IVD),
            _row_spec(IED := efeat.shape[1]),
            _full_spec((IVD, VD)), _full_spec((1, VD)),
            _full_spec((IED, ED)), _full_spec((1, ED)),
            _full_spec((VD, QD)), _full_spec((1, QD)),
            _full_spec((ED, QD)), _full_spec((1, QD)),
            _full_spec((ED, VD)), _full_spec((1, VD)),
        ],
        out_specs=[_row_spec(ED), _row_spec(QD), _row_spec(QD), _row_spec(VD)],
        out_shape=[
            jax.ShapeDtypeStruct((N_HEDGES, ED), jnp.float32),
            jax.ShapeDtypeStruct((N_NODES, QD), jnp.float32),
            jax.ShapeDtypeStruct((N_HEDGES, QD), jnp.float32),
            jax.ShapeDtypeStruct((N_HEDGES, VD), jnp.float32),
        ],
    )(vfeat, efeat,
      W_v1.T, b_v1.reshape(1, VD),
      W_e1.T, b_e1.reshape(1, ED),
      W_qv.T, b_qv.reshape(1, QD),
      W_ke.T, b_ke.reshape(1, QD),
      W_ve.T, b_ve.reshape(1, VD))

    attn, wmax = _edge_attn(k, q, src, dst)
    acc = _edge_agg(v, src, dst, attn, wmax)

    h, pred = pl.pallas_call(
        _final_body,
        grid=(grid,),
        in_specs=[
            pl.BlockSpec((_NC, _ROWS, _AW), lambda i: (0, i, 0)),
            _full_spec((VD, NCLS)), _full_spec((1, NCLS)),
        ],
        out_specs=[_row_spec(VD), _row_spec(NCLS)],
        out_shape=[
            jax.ShapeDtypeStruct((N_NODES, VD), jnp.float32),
            jax.ShapeDtypeStruct((N_NODES, NCLS), jnp.float32),
        ],
    )(acc, W_cls.T, b_cls.reshape(1, NCLS))

    return (h, feat_e, pred)


# speedup vs baseline: 3.4540x; 3.4540x over previous
import functools

import jax
import jax.numpy as jnp
from jax import lax
from jax.experimental import pallas as pl
from jax.experimental.pallas import tpu as pltpu
from jax.experimental.pallas import tpu_sc as plsc

N_NODES = 10000
N_HEDGES = 10000
N_EDGES = 320000
IVD = 128
VD = 128
QD = 64
ED = 128
NCLS = 40

_NC = 2
_NS = 16
_L = 16
_NW = _NC * _NS
_C = 256
_NCHUNK = N_EDGES // _C
_CB = 128
_NCHUNK_B = N_EDGES // _CB
_RPT = N_NODES // _NS
_AW = VD + 16

_mesh = plsc.VectorSubcoreMesh(
    core_axis_name="c", subcore_axis_name="s", num_cores=_NC, num_subcores=_NS
)
_sc_params = pltpu.CompilerParams(needs_layout_passes=False,
                                  use_tc_tiling_on_sc=False)


def _worker_id():
    return lax.axis_index("s") * _NC + lax.axis_index("c")


def _num_chunks(wid, nchunk):
    return (nchunk - 1 - wid) // _NW + 1


@functools.partial(
    pl.kernel,
    out_type=[
        jax.ShapeDtypeStruct((N_EDGES,), jnp.float32),
        jax.ShapeDtypeStruct((_NW, _L), jnp.float32),
    ],
    mesh=_mesh,
    scratch_types=[
        pltpu.VMEM((_C,), jnp.int32),
        pltpu.VMEM((_C,), jnp.int32),
        pltpu.VMEM((_C, QD), jnp.float32),
        pltpu.VMEM((_C, QD), jnp.float32),
        pltpu.VMEM((_C,), jnp.float32),
        pltpu.VMEM((_L,), jnp.float32),
        pltpu.SemaphoreType.DMA,
        pltpu.SemaphoreType.DMA,
    ],
    compiler_params=_sc_params,
)
def _edge_attn(k_hbm, q_hbm, src_hbm, dst_hbm, attn_hbm, wmax_hbm,
               srcv, dstv, kbuf, qbuf, attnv, maxv, sem0, sem1):
    wid = _worker_id()
    eidx0 = lax.iota(jnp.int32, _L)

    def chunk_body(i, macc):
        base = (wid + i * _NW) * _C
        pltpu.sync_copy(src_hbm.at[pl.ds(base, _C)], srcv)
        pltpu.sync_copy(dst_hbm.at[pl.ds(base, _C)], dstv)
        cp0 = pltpu.async_copy(k_hbm.at[srcv], kbuf, sem0)
        cp1 = pltpu.async_copy(q_hbm.at[dstv], qbuf, sem1)
        cp0.wait()
        cp1.wait()

        def grp_body(g, macc):
            e0 = g * _L
            ei = eidx0 + e0
            acc = jnp.zeros((_L,), jnp.float32)
            for d in range(QD):
                dv = jnp.full((_L,), d, jnp.int32)
                acc = acc + (plsc.load_gather(kbuf, [ei, dv]) *
                             plsc.load_gather(qbuf, [ei, dv]))
            a = jnp.where(acc >= 0.0, acc, acc * 0.01) * 0.125
            attnv[pl.ds(e0, _L)] = a
            return jnp.maximum(macc, a)

        macc = lax.fori_loop(0, _C // _L, grp_body, macc)
        pltpu.sync_copy(attnv, attn_hbm.at[pl.ds(base, _C)])
        return macc

    macc = lax.fori_loop(0, _num_chunks(wid, _NCHUNK), chunk_body,
                         jnp.full((_L,), -1e30, jnp.float32))
    maxv[...] = macc
    pltpu.sync_copy(maxv, wmax_hbm.at[wid])


@functools.partial(
    pl.kernel,
    out_type=jax.ShapeDtypeStruct((_NC, N_NODES, _AW), jnp.float32),
    mesh=_mesh,
    scratch_types=[
        pltpu.VMEM((_CB,), jnp.int32),
        pltpu.VMEM((_CB,), jnp.int32),
        pltpu.VMEM((_CB,), jnp.float32),
        pltpu.VMEM((_CB, VD), jnp.float32),
        pltpu.VMEM((_CB, _AW), jnp.float32),
        pltpu.VMEM((_NW, _L), jnp.float32),
        pltpu.VMEM_SHARED((N_NODES, _AW), jnp.float32),
        pltpu.SemaphoreType.DMA,
    ],
    compiler_params=_sc_params,
)
def _edge_agg(v_hbm, src_hbm, dst_hbm, attn_hbm, wmax_hbm, acc_hbm,
              srcv, dstv, attnv, vbuf, msgbuf, wmaxv, accum, sem0):
    cid = lax.axis_index("c")
    sid = lax.axis_index("s")
    wid = sid * _NC + cid
    eidx0 = lax.iota(jnp.int32, _L)

    pltpu.sync_copy(wmax_hbm, wmaxv)
    m = wmaxv[0]
    for r in range(1, _NW):
        m = jnp.maximum(m, wmaxv[r])
    gmax = jnp.max(m)

    zero16 = jnp.zeros((_L,), jnp.float32)

    def zrow(e, carry):
        for j in range(_AW // _L):
            msgbuf[e, pl.ds(j * _L, _L)] = zero16
        return carry

    lax.fori_loop(0, _CB, zrow, 0)
    row0 = sid * _RPT
    _zc = [(o, min(_CB, _RPT - o)) for o in range(0, _RPT, _CB)]
    for off, sz in _zc:
        pltpu.sync_copy(msgbuf.at[pl.ds(0, sz)], accum.at[pl.ds(row0 + off, sz)])
    plsc.subcore_barrier()

    col_ex = jnp.full((_L,), VD, jnp.int32)

    def chunk_body(i, carry):
        base = (wid + i * _NW) * _CB
        pltpu.sync_copy(src_hbm.at[pl.ds(base, _CB)], srcv)
        pltpu.sync_copy(dst_hbm.at[pl.ds(base, _CB)], dstv)
        pltpu.sync_copy(attn_hbm.at[pl.ds(base, _CB)], attnv)
        cp = pltpu.async_copy(v_hbm.at[srcv], vbuf, sem0)
        cp.wait()

        def grp_body(g, carry):
            e0 = g * _L
            ei = eidx0 + e0
            ex = jnp.exp(attnv[pl.ds(e0, _L)] - gmax)
            plsc.store_scatter(msgbuf, [ei, col_ex], ex)
            for d in range(VD):
                dv = jnp.full((_L,), d, jnp.int32)
                col = plsc.load_gather(vbuf, [ei, dv]) * ex
                plsc.store_scatter(msgbuf, [ei, dv], col)
            return carry

        lax.fori_loop(0, _CB // _L, grp_body, 0)
        pltpu.sync_copy(msgbuf, accum.at[dstv], add=True)
        return carry

    lax.fori_loop(0, _num_chunks(wid, _NCHUNK_B), chunk_body, 0)
    plsc.subcore_barrier()

    for off, sz in _zc:
        pltpu.sync_copy(accum.at[pl.ds(row0 + off, sz)],
                        acc_hbm.at[cid, pl.ds(row0 + off, sz)])


_ROWS = 1000


def _proj_body(vf, ef, wv1t, bv1, we1t, be1, wqvt, bqv, wket, bke, wvet, bve,
               fe_o, q_o, k_o, v_o):
    fv = jnp.dot(vf[...], wv1t[...], preferred_element_type=jnp.float32) + bv1[...]
    fe = jnp.dot(ef[...], we1t[...], preferred_element_type=jnp.float32) + be1[...]
    fe_o[...] = fe
    q_o[...] = jnp.dot(fv, wqvt[...], preferred_element_type=jnp.float32) + bqv[...]
    k_o[...] = jnp.dot(fe, wket[...], preferred_element_type=jnp.float32) + bke[...]
    v_o[...] = jnp.dot(fe, wvet[...], preferred_element_type=jnp.float32) + bve[...]


def _final_body(acc, wclst, bcls, h_o, pred_o):
    s = acc[0] + acc[1]
    denom = s[:, VD:VD + 1]
    h = s[:, :VD] / (denom + 1e-9)
    h_o[...] = h
    pred_o[...] = jnp.dot(h, wclst[...], preferred_element_type=jnp.float32) + bcls[...]


def _row_spec(cols):
    return pl.BlockSpec((_ROWS, cols), lambda i: (i, 0))


def _full_spec(shape):
    nd = len(shape)
    return pl.BlockSpec(shape, lambda i: (0,) * nd)


def kernel(vfeat, efeat, edge_index, W_v1, b_v1, W_e1, b_e1, W_qv, b_qv,
           W_ke, b_ke, W_ve, b_ve, W_cls, b_cls, first_layer, last_layer):
    src = edge_index[0]
    dst = edge_index[1]
    grid = N_NODES // _ROWS

    feat_e, q, k, v = pl.pallas_call(
        _proj_body,
        grid=(grid,),
        in_specs=[
            _row_spec(IVD),
            _row_spec(IED := efeat.shape[1]),
            _full_spec((IVD, VD)), _full_spec((1, VD)),
            _full_spec((IED, ED)), _full_spec((1, ED)),
            _full_spec((VD, QD)), _full_spec((1, QD)),
            _full_spec((ED, QD)), _full_spec((1, QD)),
            _full_spec((ED, VD)), _full_spec((1, VD)),
        ],
        out_specs=[_row_spec(ED), _row_spec(QD), _row_spec(QD), _row_spec(VD)],
        out_shape=[
            jax.ShapeDtypeStruct((N_HEDGES, ED), jnp.float32),
            jax.ShapeDtypeStruct((N_NODES, QD), jnp.float32),
            jax.ShapeDtypeStruct((N_HEDGES, QD), jnp.float32),
            jax.ShapeDtypeStruct((N_HEDGES, VD), jnp.float32),
        ],
    )(vfeat, efeat,
      W_v1.T, b_v1.reshape(1, VD),
      W_e1.T, b_e1.reshape(1, ED),
      W_qv.T, b_qv.reshape(1, QD),
      W_ke.T, b_ke.reshape(1, QD),
      W_ve.T, b_ve.reshape(1, VD))

    attn, wmax = _edge_attn(k, q, src, dst)
    acc = _edge_agg(v, src, dst, attn, wmax)

    h, pred = pl.pallas_call(
        _final_body,
        grid=(grid,),
        in_specs=[
            pl.BlockSpec((_NC, _ROWS, _AW), lambda i: (0, i, 0)),
            _full_spec((VD, NCLS)), _full_spec((1, NCLS)),
        ],
        out_specs=[_row_spec(VD), _row_spec(NCLS)],
        out_shape=[
            jax.ShapeDtypeStruct((N_NODES, VD), jnp.float32),
            jax.ShapeDtypeStruct((N_NODES, NCLS), jnp.float32),
        ],
    )(acc, W_cls.T, b_cls.reshape(1, NCLS))

    return (h, feat_e, pred)
